# baseline (speedup 1.0000x reference)
import functools

import jax
import jax.numpy as jnp
from jax import lax
from jax.experimental import pallas as pl
from jax.experimental.pallas import tpu as pltpu
from jax.experimental.pallas import tpu_sc as plsc


def _mixture_select(probs, p, params_flat, K, B, D, rows_per, NC):
    mesh = plsc.VectorSubcoreMesh(core_axis_name="c", subcore_axis_name="s")

    @functools.partial(
        pl.kernel,
        out_type=jax.ShapeDtypeStruct((B, D), jnp.float32),
        mesh=mesh,
        scratch_types=[
            pltpu.VMEM((16,), jnp.float32),
            pltpu.VMEM((16,), jnp.float32),
        ],
    )
    def run(probs_hbm, p_hbm, params_hbm, out_hbm, probs_v, p_v):
        wid = lax.axis_index("s") * NC + lax.axis_index("c")
        pltpu.sync_copy(probs_hbm, probs_v.at[pl.ds(0, K)])
        pltpu.sync_copy(p_hbm, p_v.at[pl.ds(0, 1)])
        pv = p_v[...][0]
        v = probs_v[...]
        acc = jnp.float32(0.0)
        idx = jnp.int32(0)
        for k in range(K):
            acc = acc + v[k]
            idx = idx + jnp.where(acc < pv, jnp.int32(1), jnp.int32(0))
        idx = jnp.minimum(idx, jnp.int32(K - 1))
        src_row = idx * B + wid * rows_per
        dst_row = wid * rows_per
        pltpu.sync_copy(
            params_hbm.at[pl.ds(src_row, rows_per)],
            out_hbm.at[pl.ds(dst_row, rows_per)],
        )

    return run(probs, p, params_flat)


def kernel(probs, p, params, batch_size):
    K, B, D = params.shape
    info = plsc.get_sparse_core_info()
    NC, NS = info.num_cores, info.num_subcores
    NW = NC * NS
    rows_per = B // NW
    params_flat = params.reshape(K * B, D)
    return _mixture_select(probs, p, params_flat, K, B, D, rows_per, NC)
